# baseline (speedup 1.0000x reference)
import functools
import math

import jax
import jax.numpy as jnp
from jax.experimental import pallas as pl
from jax.experimental.pallas import tpu as pltpu

_FACTOR = 0.0005


def _one_head(q, ks, k, v, n_top, L_K, L, D):
    S = jax.lax.dot_general(q, ks, (((1,), (1,)), ((), ())),
                            preferred_element_type=jnp.float32)
    M = jnp.max(S, axis=1) - jnp.sum(S, axis=1) / L_K
    R = 128
    Mv0 = M.reshape(L // R, R)
    flat_iota = (jax.lax.broadcasted_iota(jnp.int32, (L // R, R), 0) * R
                 + jax.lax.broadcasted_iota(jnp.int32, (L // R, R), 1))

    def body(i, carry):
        Mv, idxs = carry
        m = jnp.max(Mv)
        idx = jnp.min(jnp.where(Mv == m, flat_iota, L))
        Mv = jnp.where(flat_iota == idx, -jnp.inf, Mv)
        sel_iota = jax.lax.broadcasted_iota(jnp.int32, (n_top,), 0)
        idxs = jnp.where(sel_iota == i, idx, idxs)
        return Mv, idxs

    _, idxs = jax.lax.fori_loop(
        0, n_top, body, (Mv0, jnp.zeros((n_top,), jnp.int32)))

    q_iota = jax.lax.broadcasted_iota(jnp.int32, (n_top, L), 1)
    onehot = (q_iota == idxs[:, None]).astype(jnp.float32)
    sel_q = jnp.dot(onehot, q, preferred_element_type=jnp.float32)
    scores = jax.lax.dot_general(sel_q, k, (((1,), (1,)), ((), ())),
                                 preferred_element_type=jnp.float32)
    scores = scores * (1.0 / math.sqrt(D))
    scores = scores - jnp.max(scores, axis=1, keepdims=True)
    w = jnp.exp(scores)
    w = w / jnp.sum(w, axis=1, keepdims=True)
    ctx = jnp.dot(w, v, preferred_element_type=jnp.float32)

    return jax.lax.dot_general(onehot, ctx, (((0,), (0,)), ((), ())),
                               preferred_element_type=jnp.float32)


def _fused_body(q_ref, ks_ref, k_ref, v_ref, o_ref, *, n_top, L_K, D):
    L = q_ref.shape[1]
    for j in range(q_ref.shape[2] // D):
        sl = slice(j * D, (j + 1) * D)
        o_ref[0, :, sl] = _one_head(
            q_ref[0, :, sl], ks_ref[0, :, sl], k_ref[0, :, sl], v_ref[0, :, sl],
            n_top, L_K, L, D)


def kernel(queries, keys, values):
    B, L, H, D = queries.shape
    L_K = keys.shape[1]
    n_top = max(int(L * _FACTOR * math.log(L_K)), 1)
    U_part = min(n_top, L_K)
    perm = jax.random.permutation(jax.random.key(42), L_K)[:U_part]
    k_sample = keys[:, perm, :, :]

    HP = 128 // D if D < 128 else 1
    q3 = queries.reshape(B, L, H * D)
    k3 = keys.reshape(B, L, H * D)
    v3 = values.reshape(B, L, H * D)
    ks3 = k_sample.reshape(B, U_part, H * D)

    grid = (B, H // HP)
    spec = pl.BlockSpec((1, L, HP * D), lambda b, h: (b, 0, h))
    ksspec = pl.BlockSpec((1, U_part, HP * D), lambda b, h: (b, 0, h))
    body = functools.partial(_fused_body, n_top=n_top, L_K=L_K, D=D)
    out = pl.pallas_call(
        body,
        grid=grid,
        in_specs=[spec, ksspec, spec, spec],
        out_specs=spec,
        out_shape=jax.ShapeDtypeStruct((B, L, H * D), jnp.float32),
        compiler_params=pltpu.CompilerParams(
            dimension_semantics=("parallel", "parallel"),
        ),
    )(q3, ks3, k3, v3)
    return out.reshape(B, L, H, D)
